# Initial kernel scaffold; baseline (speedup 1.0000x reference)
import functools

import jax
import jax.numpy as jnp
from jax import lax
from jax.experimental import pallas as pl
from jax.experimental.pallas import tpu as pltpu
from jax.experimental.pallas import tpu_sc as plsc

N_FIELDS = 26
N_COLS = N_FIELDS + 2
FIELD_VOCAB = 100000
VEC_DIM = 16
BATCH = 16384
TOTAL_DIM = N_COLS * VEC_DIM

_INFO = plsc.get_sparse_core_info()
NW = _INFO.num_cores * _INFO.num_subcores
BPW = BATCH // NW
CHUNK = 128
NCHUNK = BPW // CHUNK


@functools.partial(
    pl.kernel,
    out_type=jax.ShapeDtypeStruct((BATCH, TOTAL_DIM), jnp.float32),
    mesh=plsc.VectorSubcoreMesh(core_axis_name="c", subcore_axis_name="s"),
    scratch_types=[
        pltpu.VMEM((N_COLS, BPW), jnp.int32),
        pltpu.VMEM((BPW, VEC_DIM), jnp.float32),
        pltpu.SemaphoreType.DMA,
    ],
)
def _gather(user_hbm, item_hbm, feat_hbm, idx_hbm, out_hbm, idx_v, rows_v, sem):
    wid = lax.axis_index("s") * _INFO.num_cores + lax.axis_index("c")
    base = wid * BPW
    pltpu.sync_copy(idx_hbm.at[:, pl.ds(base, BPW)], idx_v)

    def gather_field(table, f):
        copies = []
        for ch in range(NCHUNK):
            copies.append(
                pltpu.async_copy(
                    table.at[idx_v.at[f, pl.ds(ch * CHUNK, CHUNK)]],
                    rows_v.at[pl.ds(ch * CHUNK, CHUNK)],
                    sem,
                )
            )
        for cp in copies:
            cp.wait()
        pltpu.sync_copy(
            rows_v, out_hbm.at[pl.ds(base, BPW), pl.ds(f * VEC_DIM, VEC_DIM)]
        )

    gather_field(user_hbm, 0)
    gather_field(item_hbm, 1)

    def body(f, carry):
        gather_field(feat_hbm, f)
        return carry

    lax.fori_loop(2, N_COLS, body, 0)


def _fm_body(v_ref, k_ref, w_ref, b_ref, o_ref):
    v = v_ref[...]
    k = k_ref[...]
    s = jnp.dot(v, k, preferred_element_type=jnp.float32)
    q = jnp.dot(v * v, k * k, preferred_element_type=jnp.float32)
    liner = jnp.dot(v, w_ref[...], preferred_element_type=jnp.float32)
    cross = 0.5 * jnp.sum(s * s - q, axis=1, keepdims=True)
    z = liner + b_ref[0, 0] + cross
    o_ref[...] = 1.0 / (1.0 + jnp.exp(-z))


_BB = 2048


def _fm(vec, k_mat, w, b2):
    return pl.pallas_call(
        _fm_body,
        grid=(BATCH // _BB,),
        in_specs=[
            pl.BlockSpec((_BB, TOTAL_DIM), lambda i: (i, 0)),
            pl.BlockSpec((TOTAL_DIM, VEC_DIM), lambda i: (0, 0)),
            pl.BlockSpec((TOTAL_DIM, 1), lambda i: (0, 0)),
            pl.BlockSpec(memory_space=pltpu.SMEM),
        ],
        out_specs=pl.BlockSpec((_BB, 1), lambda i: (i, 0)),
        out_shape=jax.ShapeDtypeStruct((BATCH, 1), jnp.float32),
    )(vec, k_mat, w, b2)


def kernel(inputs, user_table, item_table, feat_tables, w, b, k_mat):
    idx = inputs.astype(jnp.int32)
    offs = jnp.concatenate(
        [
            jnp.zeros((2,), jnp.int32),
            jnp.arange(N_FIELDS, dtype=jnp.int32) * FIELD_VOCAB,
        ]
    )
    idx_t = idx.T + offs[:, None]
    feat_flat = feat_tables.reshape(N_FIELDS * FIELD_VOCAB, VEC_DIM)
    vec = _gather(user_table, item_table, feat_flat, idx_t)
    return _fm(vec, k_mat, w, b.reshape(1, 1))


# speedup vs baseline: 1.0374x; 1.0374x over previous
import jax
import jax.numpy as jnp
from jax import lax
from jax.experimental import pallas as pl
from jax.experimental.pallas import tpu as pltpu

N_FIELDS = 26
N_COLS = N_FIELDS + 2
VEC_DIM = 16
BATCH = 16384
TOTAL_DIM = N_COLS * VEC_DIM

_BB = 2048


def _fm_body(*refs):
    v_refs = refs[:N_COLS]
    k_ref, w_ref, b_ref, o_ref = refs[N_COLS:]
    v = jnp.concatenate([r[...] for r in v_refs], axis=0)
    k = k_ref[...]
    w = w_ref[...]
    dn = (((0,), (0,)), ((), ()))
    s = lax.dot_general(k, v, dn, preferred_element_type=jnp.float32)
    q = lax.dot_general(k * k, v * v, dn, preferred_element_type=jnp.float32)
    liner = lax.dot_general(w, v, dn, preferred_element_type=jnp.float32)
    cross = 0.5 * jnp.sum(s * s - q, axis=0, keepdims=True)
    z = liner + b_ref[0, 0] + cross
    o_ref[...] = 1.0 / (1.0 + jnp.exp(-z))


def _fm(vts, k_mat, w, b2):
    return pl.pallas_call(
        _fm_body,
        grid=(BATCH // _BB,),
        in_specs=[
            pl.BlockSpec((VEC_DIM, _BB), lambda i: (0, i))
            for _ in range(N_COLS)
        ]
        + [
            pl.BlockSpec((TOTAL_DIM, VEC_DIM), lambda i: (0, 0)),
            pl.BlockSpec((TOTAL_DIM, 1), lambda i: (0, 0)),
            pl.BlockSpec(memory_space=pltpu.SMEM),
        ],
        out_specs=pl.BlockSpec((1, _BB), lambda i: (0, i)),
        out_shape=jax.ShapeDtypeStruct((1, BATCH), jnp.float32),
    )(*vts, k_mat, w, b2)


def kernel(inputs, user_table, item_table, feat_tables, w, b, k_mat):
    gathered = [
        jnp.take(user_table, inputs[:, 0], axis=0),
        jnp.take(item_table, inputs[:, 1], axis=0),
    ]
    for i in range(N_FIELDS):
        gathered.append(jnp.take(feat_tables[i], inputs[:, i + 2], axis=0))
    vts = [g.T for g in gathered]
    yt = _fm(vts, k_mat, w, b.reshape(1, 1))
    return yt.reshape(BATCH, 1)


# speedup vs baseline: 3.7915x; 3.6550x over previous
import jax
import jax.numpy as jnp
from jax import lax
from jax.experimental import pallas as pl
from jax.experimental.pallas import tpu as pltpu

N_FIELDS = 26
N_COLS = N_FIELDS + 2
FIELD_VOCAB = 100000
VEC_DIM = 16
BATCH = 16384
TOTAL_DIM = N_COLS * VEC_DIM

_BB = 4096
_NB = BATCH // _BB


def _fm_body(*refs):
    v_refs = refs[:N_COLS]
    k_ref, w_ref, b_ref, o_ref = refs[N_COLS:]
    v = jnp.concatenate([r[...] for r in v_refs], axis=0)
    k = k_ref[...]
    w = w_ref[...]
    dn = (((0,), (0,)), ((), ()))
    s = lax.dot_general(k, v, dn, preferred_element_type=jnp.float32)
    q = lax.dot_general(k * k, v * v, dn, preferred_element_type=jnp.float32)
    liner = lax.dot_general(w, v, dn, preferred_element_type=jnp.float32)
    cross = 0.5 * jnp.sum(s * s - q, axis=0, keepdims=True)
    z = liner + b_ref[0, 0] + cross
    o_ref[...] = 1.0 / (1.0 + jnp.exp(-z))


def _fm(user_t, item_t, feat_t, k_mat, w, b2):
    in_specs = [
        pl.BlockSpec((VEC_DIM, _BB), lambda i: (0, i)),
        pl.BlockSpec((VEC_DIM, _BB), lambda i: (0, i)),
    ]
    for f in range(N_FIELDS):
        in_specs.append(
            pl.BlockSpec((VEC_DIM, _BB), lambda i, f=f: (0, f * _NB + i))
        )
    in_specs += [
        pl.BlockSpec((TOTAL_DIM, VEC_DIM), lambda i: (0, 0)),
        pl.BlockSpec((TOTAL_DIM, 1), lambda i: (0, 0)),
        pl.BlockSpec(memory_space=pltpu.SMEM),
    ]
    return pl.pallas_call(
        _fm_body,
        grid=(_NB,),
        in_specs=in_specs,
        out_specs=pl.BlockSpec((1, _BB), lambda i: (0, i)),
        out_shape=jax.ShapeDtypeStruct((1, BATCH), jnp.float32),
    )(user_t, item_t, *([feat_t] * N_FIELDS), k_mat, w, b2)


def kernel(inputs, user_table, item_table, feat_tables, w, b, k_mat):
    g_user = jnp.take(user_table, inputs[:, 0], axis=0, mode="clip")
    g_item = jnp.take(item_table, inputs[:, 1], axis=0, mode="clip")
    vocab_ids = inputs[:, 2:].T.astype(jnp.int32)
    field_ids = jax.lax.broadcasted_iota(jnp.int32, (N_FIELDS, BATCH), 0)
    starts = jnp.stack([field_ids, vocab_ids], axis=-1).reshape(-1, 2)
    dnums = lax.GatherDimensionNumbers(
        offset_dims=(1,),
        collapsed_slice_dims=(0, 1),
        start_index_map=(0, 1),
    )
    g_feat = lax.gather(
        feat_tables,
        starts,
        dnums,
        slice_sizes=(1, 1, VEC_DIM),
        mode=lax.GatherScatterMode.PROMISE_IN_BOUNDS,
    )
    yt = _fm(
        g_user.T, g_item.T, g_feat.T, k_mat, w, b.reshape(1, 1)
    )
    return yt.reshape(BATCH, 1)


# speedup vs baseline: 3.8093x; 1.0047x over previous
import jax
import jax.numpy as jnp
from jax import lax
from jax.experimental import pallas as pl
from jax.experimental.pallas import tpu as pltpu

N_FIELDS = 26
N_COLS = N_FIELDS + 2
FIELD_VOCAB = 100000
VEC_DIM = 16
BATCH = 16384
TOTAL_DIM = N_COLS * VEC_DIM

_BB = 8192
_NB = BATCH // _BB


def _fm_body(*refs):
    v_refs = refs[:N_COLS]
    k_ref, w_ref, b_ref, o_ref = refs[N_COLS:]
    v = jnp.concatenate([r[...] for r in v_refs], axis=0)
    k = k_ref[...]
    w = w_ref[...]
    dn = (((0,), (0,)), ((), ()))
    s = lax.dot_general(k, v, dn, preferred_element_type=jnp.float32)
    q = lax.dot_general(k * k, v * v, dn, preferred_element_type=jnp.float32)
    liner = lax.dot_general(w, v, dn, preferred_element_type=jnp.float32)
    cross = 0.5 * jnp.sum(s * s - q, axis=0, keepdims=True)
    z = liner + b_ref[0, 0] + cross
    o_ref[...] = 1.0 / (1.0 + jnp.exp(-z))


def _fm(user_t, item_t, feat_t, k_mat, w, b2):
    in_specs = [
        pl.BlockSpec((VEC_DIM, _BB), lambda i: (0, i)),
        pl.BlockSpec((VEC_DIM, _BB), lambda i: (0, i)),
    ]
    for f in range(N_FIELDS):
        in_specs.append(
            pl.BlockSpec((VEC_DIM, _BB), lambda i, f=f: (0, f * _NB + i))
        )
    in_specs += [
        pl.BlockSpec((TOTAL_DIM, VEC_DIM), lambda i: (0, 0)),
        pl.BlockSpec((TOTAL_DIM, 1), lambda i: (0, 0)),
        pl.BlockSpec(memory_space=pltpu.SMEM),
    ]
    return pl.pallas_call(
        _fm_body,
        grid=(_NB,),
        in_specs=in_specs,
        out_specs=pl.BlockSpec((1, _BB), lambda i: (0, i)),
        out_shape=jax.ShapeDtypeStruct((1, BATCH), jnp.float32),
    )(user_t, item_t, *([feat_t] * N_FIELDS), k_mat, w, b2)


def kernel(inputs, user_table, item_table, feat_tables, w, b, k_mat):
    g_user = jnp.take(user_table, inputs[:, 0], axis=0, mode="clip")
    g_item = jnp.take(item_table, inputs[:, 1], axis=0, mode="clip")
    vocab_ids = inputs[:, 2:].T.astype(jnp.int32)
    field_ids = jax.lax.broadcasted_iota(jnp.int32, (N_FIELDS, BATCH), 0)
    starts = jnp.stack([field_ids, vocab_ids], axis=-1).reshape(-1, 2)
    dnums = lax.GatherDimensionNumbers(
        offset_dims=(1,),
        collapsed_slice_dims=(0, 1),
        start_index_map=(0, 1),
    )
    g_feat = lax.gather(
        feat_tables,
        starts,
        dnums,
        slice_sizes=(1, 1, VEC_DIM),
        mode=lax.GatherScatterMode.PROMISE_IN_BOUNDS,
    )
    yt = _fm(
        g_user.T, g_item.T, g_feat.T, k_mat, w, b.reshape(1, 1)
    )
    return yt.reshape(BATCH, 1)
